# baseline (speedup 1.0000x reference)
import functools

import jax
import jax.numpy as jnp
from jax import lax
from jax.experimental import pallas as pl
from jax.experimental.pallas import tpu as pltpu
from jax.experimental.pallas import tpu_sc as plsc

_EPS = 1e-08
_N = 3276800
_NC = 2
_NS = 16
_NW = _NC * _NS
_PER_W = _N // _NW
_CHUNK = 20480
_NCHUNK = _PER_W // _CHUNK
_L = 16


def _body(x_hbm, g_hbm, mins_hbm, maxs_hbm, out_hbm,
          scale_v, offs_v, xb, gb, ob):
    wid = lax.axis_index("s") * _NC + lax.axis_index("c")

    pltpu.sync_copy(mins_hbm, scale_v)
    pltpu.sync_copy(maxs_hbm, offs_v)
    m = scale_v[...]
    M = offs_v[...]
    sc = (1.0 - 2.0 * _EPS) / (M - m)
    scale_v[...] = sc
    offs_v[...] = _EPS - m * sc

    base = wid * _PER_W

    def chunk_body(ci, carry):
        off = base + ci * _CHUNK
        pltpu.sync_copy(x_hbm.at[pl.ds(off, _CHUNK)], xb)
        pltpu.sync_copy(g_hbm.at[pl.ds(off, _CHUNK)], gb)

        def vec_body(i, c2):
            s = pl.ds(i * _L, _L)
            idx = gb[s] - 1
            sg = plsc.load_gather(scale_v, [idx])
            og = plsc.load_gather(offs_v, [idx])
            ob[s] = xb[s] * sg + og
            return c2

        lax.fori_loop(0, _CHUNK // _L, vec_body, 0)
        pltpu.sync_copy(ob, out_hbm.at[pl.ds(off, _CHUNK)])
        return carry

    lax.fori_loop(0, _NCHUNK, chunk_body, 0)


@jax.jit
def _run(x, group, mins, maxs):
    mesh = plsc.VectorSubcoreMesh(core_axis_name="c", subcore_axis_name="s")
    kern = functools.partial(
        pl.kernel,
        mesh=mesh,
        compiler_params=pltpu.CompilerParams(needs_layout_passes=False),
        out_type=jax.ShapeDtypeStruct((_N,), jnp.float32),
        scratch_types=[
            pltpu.VMEM((_L,), jnp.float32),
            pltpu.VMEM((_L,), jnp.float32),
            pltpu.VMEM((_CHUNK,), jnp.float32),
            pltpu.VMEM((_CHUNK,), jnp.int32),
            pltpu.VMEM((_CHUNK,), jnp.float32),
        ],
    )(_body)
    return kern(x, group, mins, maxs)


def kernel(x, group, mins, maxs):
    return _run(x, group, mins, maxs)
